# baseline (speedup 1.0000x reference)
import jax
import jax.numpy as jnp
from jax.experimental import pallas as pl
from jax.experimental.pallas import tpu as pltpu

SUB_BLOCK = 1024
NUM_STREAMS = 4
TOKEN_BLOCK = SUB_BLOCK * NUM_STREAMS


def _gating_block(*refs):
    x_refs = refs[:NUM_STREAMS]
    w1t_ref, b1_ref, w2t_ref, b2_ref, out_ref = refs[NUM_STREAMS:]
    for j in range(NUM_STREAMS):
        xb = x_refs[j][...].astype(jnp.bfloat16)
        h = jnp.tanh(
            jnp.dot(xb, w1t_ref[...], preferred_element_type=jnp.float32)
            + b1_ref[...]
        )
        out_ref[j * SUB_BLOCK:(j + 1) * SUB_BLOCK, :] = (
            jnp.dot(h.astype(jnp.bfloat16), w2t_ref[...],
                    preferred_element_type=jnp.float32)
            + b2_ref[...]
        )


@jax.jit
def _gating(x, w1t, b1, w2t, b2):
    tokens, feats = x.shape
    num_experts = w1t.shape[1]
    grid = (tokens // TOKEN_BLOCK,)

    def x_spec(j):
        return pl.BlockSpec((SUB_BLOCK, feats),
                            lambda i, j=j: (NUM_STREAMS * i + j, 0))

    gates = pl.pallas_call(
        _gating_block,
        grid=grid,
        in_specs=[x_spec(j) for j in range(NUM_STREAMS)] + [
            pl.BlockSpec((feats, num_experts), lambda i: (0, 0)),
            pl.BlockSpec((1, num_experts), lambda i: (0, 0)),
            pl.BlockSpec((num_experts, num_experts), lambda i: (0, 0)),
            pl.BlockSpec((1, num_experts), lambda i: (0, 0)),
        ],
        out_specs=pl.BlockSpec((TOKEN_BLOCK, num_experts), lambda i: (i, 0)),
        out_shape=jax.ShapeDtypeStruct((tokens, num_experts), jnp.float32),
        compiler_params=pltpu.CompilerParams(
            dimension_semantics=("parallel",),
        ),
    )(*([x] * NUM_STREAMS), w1t, b1, w2t, b2)
    return gates


def kernel(x, W1, b1, W2, b2, train):
    w1t = W1.T.astype(jnp.bfloat16)
    w2t = W2.T.astype(jnp.bfloat16)
    gates = _gating(x, w1t, b1.reshape(1, -1), w2t, b2.reshape(1, -1))
    return (gates, gates)


# speedup vs baseline: 1.0680x; 1.0680x over previous
import jax
import jax.numpy as jnp
from jax.experimental import pallas as pl
from jax.experimental.pallas import tpu as pltpu

TOKEN_BLOCK = 4096


def _gating_block(x_ref, w1t_ref, b1_ref, w2t_ref, b2_ref, out_ref):
    xb = x_ref[...].astype(jnp.bfloat16)
    h = jnp.tanh(
        jnp.dot(xb, w1t_ref[...], preferred_element_type=jnp.float32)
        + b1_ref[...]
    )
    out_ref[...] = (
        jnp.dot(h.astype(jnp.bfloat16), w2t_ref[...],
                preferred_element_type=jnp.float32)
        + b2_ref[...]
    )


@jax.jit
def _gating(x, w1t, b1, w2t, b2):
    tokens = x.shape[0]
    num_experts = w1t.shape[1]
    grid = (tokens // TOKEN_BLOCK,)
    gates = pl.pallas_call(
        _gating_block,
        grid=grid,
        in_specs=[
            pl.BlockSpec((TOKEN_BLOCK, x.shape[1]), lambda i: (i, 0)),
            pl.BlockSpec((x.shape[1], num_experts), lambda i: (0, 0)),
            pl.BlockSpec((1, num_experts), lambda i: (0, 0)),
            pl.BlockSpec((num_experts, num_experts), lambda i: (0, 0)),
            pl.BlockSpec((1, num_experts), lambda i: (0, 0)),
        ],
        out_specs=pl.BlockSpec((TOKEN_BLOCK, num_experts), lambda i: (i, 0)),
        out_shape=jax.ShapeDtypeStruct((tokens, num_experts), jnp.float32),
        compiler_params=pltpu.CompilerParams(
            dimension_semantics=("parallel",),
        ),
    )(x, w1t, b1, w2t, b2)
    return gates


def kernel(x, W1, b1, W2, b2, train):
    w1t = W1.T.astype(jnp.bfloat16)
    w2t = W2.T.astype(jnp.bfloat16)
    gates = _gating(x, w1t, b1.reshape(1, -1), w2t, b2.reshape(1, -1))
    return (gates, gates)


# speedup vs baseline: 1.4722x; 1.3785x over previous
import jax
import jax.numpy as jnp
from jax.experimental import pallas as pl
from jax.experimental.pallas import tpu as pltpu

TOKEN_BLOCK = 4096


def _gating_block(x_ref, w1_ref, b1_ref, w2_ref, b2_ref, out_ref):
    xb = x_ref[...].astype(jnp.bfloat16)
    w1b = w1_ref[...].astype(jnp.bfloat16)
    h_t = jnp.tanh(
        jax.lax.dot_general(w1b, xb, (((1,), (1,)), ((), ())),
                            preferred_element_type=jnp.float32)
        + b1_ref[...]
    )
    w2b = w2_ref[...].astype(jnp.bfloat16)
    out_ref[...] = (
        jax.lax.dot_general(w2b, h_t.astype(jnp.bfloat16),
                            (((1,), (0,)), ((), ())),
                            preferred_element_type=jnp.float32)
        + b2_ref[...]
    )


@jax.jit
def _gating(x, w1, b1, w2, b2):
    tokens, feats = x.shape
    num_experts = w1.shape[0]
    grid = (tokens // TOKEN_BLOCK,)
    gates_t = pl.pallas_call(
        _gating_block,
        grid=grid,
        in_specs=[
            pl.BlockSpec((TOKEN_BLOCK, feats), lambda i: (i, 0)),
            pl.BlockSpec((num_experts, feats), lambda i: (0, 0)),
            pl.BlockSpec((num_experts, 1), lambda i: (0, 0)),
            pl.BlockSpec((num_experts, num_experts), lambda i: (0, 0)),
            pl.BlockSpec((num_experts, 1), lambda i: (0, 0)),
        ],
        out_specs=pl.BlockSpec((num_experts, TOKEN_BLOCK), lambda i: (0, i)),
        out_shape=jax.ShapeDtypeStruct((num_experts, tokens), jnp.float32),
        compiler_params=pltpu.CompilerParams(
            dimension_semantics=("parallel",),
        ),
    )(x, w1, b1, w2, b2)
    return gates_t.T


def kernel(x, W1, b1, W2, b2, train):
    gates = _gating(x, W1, b1.reshape(-1, 1), W2, b2.reshape(-1, 1))
    return (gates, gates)


# speedup vs baseline: 1.5456x; 1.0498x over previous
import jax
import jax.numpy as jnp
from jax.experimental import pallas as pl
from jax.experimental.pallas import tpu as pltpu

TOKEN_BLOCK = 4096


def _gating_block(x_ref, w1_ref, b1_ref, w2_ref, b2_ref, out_ref, out2_ref):
    xb = x_ref[...].astype(jnp.bfloat16)
    w1b = w1_ref[...].astype(jnp.bfloat16)
    h_t = jnp.tanh(
        jax.lax.dot_general(w1b, xb, (((1,), (1,)), ((), ())),
                            preferred_element_type=jnp.float32)
        + b1_ref[...]
    )
    w2b = w2_ref[...].astype(jnp.bfloat16)
    gates_t = (
        jax.lax.dot_general(w2b, h_t.astype(jnp.bfloat16),
                            (((1,), (0,)), ((), ())),
                            preferred_element_type=jnp.float32)
        + b2_ref[...]
    )
    out_ref[...] = gates_t
    out2_ref[...] = gates_t


@jax.jit
def _gating(x, w1, b1, w2, b2):
    tokens, feats = x.shape
    num_experts = w1.shape[0]
    grid = (tokens // TOKEN_BLOCK,)
    gates_t = pl.pallas_call(
        _gating_block,
        grid=grid,
        in_specs=[
            pl.BlockSpec((TOKEN_BLOCK, feats), lambda i: (i, 0)),
            pl.BlockSpec((num_experts, feats), lambda i: (0, 0)),
            pl.BlockSpec((num_experts, 1), lambda i: (0, 0)),
            pl.BlockSpec((num_experts, num_experts), lambda i: (0, 0)),
            pl.BlockSpec((num_experts, 1), lambda i: (0, 0)),
        ],
        out_specs=[
            pl.BlockSpec((num_experts, TOKEN_BLOCK), lambda i: (0, i)),
            pl.BlockSpec((num_experts, TOKEN_BLOCK), lambda i: (0, i)),
        ],
        out_shape=[
            jax.ShapeDtypeStruct((num_experts, tokens), jnp.float32),
            jax.ShapeDtypeStruct((num_experts, tokens), jnp.float32),
        ],
        compiler_params=pltpu.CompilerParams(
            dimension_semantics=("parallel",),
        ),
    )(x, w1, b1, w2, b2)
    return gates_t[0].T, gates_t[1].T


def kernel(x, W1, b1, W2, b2, train):
    out, gates = _gating(x, W1, b1.reshape(-1, 1), W2, b2.reshape(-1, 1))
    return (out, gates)
